# Initial kernel scaffold; baseline (speedup 1.0000x reference)
import jax
import jax.numpy as jnp
from jax.experimental import pallas as pl


def kernel(g, feat, cls_spec_avg_feats, W1, b1, W2, b2, W_sigma):
    raise NotImplementedError("write your pallas kernel here")


# speedup vs baseline: 3.5692x; 3.5692x over previous
import functools

import jax
import jax.numpy as jnp
from jax import lax
from jax.experimental import pallas as pl
from jax.experimental.pallas import tpu as pltpu
from jax.experimental.pallas import tpu_sc as plsc

N = 10000
E = 320000
IN_DIM = 128
HID = 256
OUT = 128

NC = 2
NS = 16
NW = NC * NS
C = 128
CP = 80
EPT = C * CP
EP = NW * EPT
NP = 10240
RPT = NP // NS

BR = 1280
GRID = NP // BR

_MESH = dict(core_axis_name="c", subcore_axis_name="s", num_cores=NC,
             num_subcores=NS)


def _deg_body(src_hbm, dst_hbm, out_hbm, src_v, dst_v, ones_v, zer_v,
              dego_sh, degi_sh):
    c = lax.axis_index("c")
    s = lax.axis_index("s")
    wid = c * NS + s
    pltpu.sync_copy(src_hbm.at[wid], src_v)
    pltpu.sync_copy(dst_hbm.at[wid], dst_v)

    def fill_ones(i, _):
        ones_v[pl.ds(i * 16, 16)] = jnp.full((16,), 1.0, jnp.float32)
        return 0
    lax.fori_loop(0, C // 16, fill_ones, 0)

    def fill_zero(i, _):
        zer_v[pl.ds(i * 16, 16)] = jnp.zeros((16,), jnp.float32)
        return 0
    lax.fori_loop(0, RPT // 16, fill_zero, 0)

    pltpu.sync_copy(zer_v, dego_sh.at[pl.ds(s * RPT, RPT)])
    pltpu.sync_copy(zer_v, degi_sh.at[pl.ds(s * RPT, RPT)])
    plsc.subcore_barrier()

    def step(j, _):
        pltpu.sync_copy(ones_v, dego_sh.at[src_v.at[j]], add=True)
        pltpu.sync_copy(ones_v, degi_sh.at[dst_v.at[j]], add=True)
        return 0
    lax.fori_loop(0, CP, step, 0)
    plsc.subcore_barrier()

    pltpu.sync_copy(dego_sh.at[pl.ds(s * RPT, RPT)],
                    out_hbm.at[c, 0, pl.ds(s * RPT, RPT)])
    pltpu.sync_copy(degi_sh.at[pl.ds(s * RPT, RPT)],
                    out_hbm.at[c, 1, pl.ds(s * RPT, RPT)])


def _agg_body(tab_hbm, src_hbm, dst_hbm, out_hbm, src_v, dst_v, buf_v,
              acc_sh, sem):
    c = lax.axis_index("c")
    s = lax.axis_index("s")
    wid = c * NS + s
    pltpu.sync_copy(src_hbm.at[wid], src_v)
    pltpu.sync_copy(dst_hbm.at[wid], dst_v)

    def zrow(r, _):
        def zcol(k, _):
            buf_v[r, pl.ds(k * 16, 16)] = jnp.zeros((16,), jnp.float32)
            return 0
        lax.fori_loop(0, IN_DIM // 16, zcol, 0)
        return 0
    lax.fori_loop(0, C, zrow, 0)

    def zcp(k, _):
        pltpu.sync_copy(buf_v, acc_sh.at[pl.ds(s * RPT + k * C, C)])
        return 0
    lax.fori_loop(0, RPT // C, zcp, 0)
    plsc.subcore_barrier()

    def step(j, _):
        pltpu.async_copy(tab_hbm.at[src_v.at[j]], buf_v, sem).wait()
        pltpu.sync_copy(buf_v, acc_sh.at[dst_v.at[j]], add=True)
        return 0
    lax.fori_loop(0, CP, step, 0)
    plsc.subcore_barrier()

    def cout(k, _):
        pltpu.sync_copy(acc_sh.at[pl.ds(s * RPT + k * C, C)],
                        out_hbm.at[c, pl.ds(s * RPT + k * C, C)])
        return 0
    lax.fori_loop(0, RPT // C, cout, 0)


@functools.cache
def _deg_call():
    return pl.kernel(
        _deg_body,
        out_type=jax.ShapeDtypeStruct((NC, 2, NP), jnp.float32),
        mesh=plsc.VectorSubcoreMesh(**_MESH),
        scratch_types=[
            pltpu.VMEM((CP, C), jnp.int32),
            pltpu.VMEM((CP, C), jnp.int32),
            pltpu.VMEM((C,), jnp.float32),
            pltpu.VMEM((RPT,), jnp.float32),
            pltpu.VMEM_SHARED((NP,), jnp.float32),
            pltpu.VMEM_SHARED((NP,), jnp.float32),
        ],
    )


@functools.cache
def _agg_call():
    return pl.kernel(
        _agg_body,
        out_type=jax.ShapeDtypeStruct((NC, NP, IN_DIM), jnp.float32),
        mesh=plsc.VectorSubcoreMesh(**_MESH),
        scratch_types=[
            pltpu.VMEM((CP, C), jnp.int32),
            pltpu.VMEM((CP, C), jnp.int32),
            pltpu.VMEM((C, IN_DIM), jnp.float32),
            pltpu.VMEM_SHARED((NP, IN_DIM), jnp.float32),
            pltpu.SemaphoreType.DMA,
        ],
    )


def _norm_m1_body(deg_ref, feat_ref, m1_ref, ns_ref, nd_ref):
    d = deg_ref[...]
    deg_o = d[:, 0] + d[:, 2]
    deg_i = d[:, 1] + d[:, 3]
    ns = jnp.where(deg_o > 0, lax.rsqrt(jnp.maximum(deg_o, 1.0)), 0.0)
    nd = jnp.where(deg_i > 0, lax.rsqrt(jnp.maximum(deg_i, 1.0)), 0.0)
    ns_ref[...] = ns[:, None]
    nd_ref[...] = nd[:, None]
    m1_ref[...] = feat_ref[...] * ns[:, None]


_norm_m1_call = pl.pallas_call(
    _norm_m1_body,
    grid=(GRID,),
    in_specs=[
        pl.BlockSpec((BR, 4), lambda i: (i, 0)),
        pl.BlockSpec((BR, IN_DIM), lambda i: (i, 0)),
    ],
    out_specs=[
        pl.BlockSpec((BR, IN_DIM), lambda i: (i, 0)),
        pl.BlockSpec((BR, 1), lambda i: (i, 0)),
        pl.BlockSpec((BR, 1), lambda i: (i, 0)),
    ],
    out_shape=[
        jax.ShapeDtypeStruct((NP, IN_DIM), jnp.float32),
        jax.ShapeDtypeStruct((NP, 1), jnp.float32),
        jax.ShapeDtypeStruct((NP, 1), jnp.float32),
    ],
)


def _mid_body(q_ref, ns_ref, nd_ref, w1_ref, b1_ref, w2_ref, h1_ref, m2_ref):
    a = (q_ref[0] + q_ref[1]) * nd_ref[...]
    h1 = jnp.dot(a, w1_ref[...], preferred_element_type=jnp.float32)
    h1 = jnp.maximum(h1 + b1_ref[...], 0.0)
    h1_ref[...] = h1
    p = jnp.dot(h1, w2_ref[...], preferred_element_type=jnp.float32)
    m2_ref[...] = p * ns_ref[...]


_mid_call = pl.pallas_call(
    _mid_body,
    grid=(GRID,),
    in_specs=[
        pl.BlockSpec((2, BR, IN_DIM), lambda i: (0, i, 0)),
        pl.BlockSpec((BR, 1), lambda i: (i, 0)),
        pl.BlockSpec((BR, 1), lambda i: (i, 0)),
        pl.BlockSpec((IN_DIM, HID), lambda i: (0, 0)),
        pl.BlockSpec((1, HID), lambda i: (0, 0)),
        pl.BlockSpec((HID, OUT), lambda i: (0, 0)),
    ],
    out_specs=[
        pl.BlockSpec((BR, HID), lambda i: (i, 0)),
        pl.BlockSpec((BR, OUT), lambda i: (i, 0)),
    ],
    out_shape=[
        jax.ShapeDtypeStruct((NP, HID), jnp.float32),
        jax.ShapeDtypeStruct((NP, OUT), jnp.float32),
    ],
)


def _fin_body(q_ref, nd_ref, b2_ref, h2_ref):
    h2_ref[...] = (q_ref[0] + q_ref[1]) * nd_ref[...] + b2_ref[...]


_fin_call = pl.pallas_call(
    _fin_body,
    grid=(GRID,),
    in_specs=[
        pl.BlockSpec((2, BR, OUT), lambda i: (0, i, 0)),
        pl.BlockSpec((BR, 1), lambda i: (i, 0)),
        pl.BlockSpec((1, OUT), lambda i: (0, 0)),
    ],
    out_specs=pl.BlockSpec((BR, OUT), lambda i: (i, 0)),
    out_shape=jax.ShapeDtypeStruct((NP, OUT), jnp.float32),
)


def _sig_body(cls_ref, ws_ref, out_ref):
    z = jnp.dot(cls_ref[...], ws_ref[...], preferred_element_type=jnp.float32)
    out_ref[...] = jnp.maximum(z, 0.0) + jnp.log(1.0 + jnp.exp(-jnp.abs(z)))


_sig_call = pl.pallas_call(
    _sig_body,
    out_shape=jax.ShapeDtypeStruct((10, 1), jnp.float32),
)


def kernel(g, feat, cls_spec_avg_feats, W1, b1, W2, b2, W_sigma):
    src, dst = g[0], g[1]
    padv = jnp.full((EP - E,), N, dtype=jnp.int32)
    src3 = jnp.concatenate([src, padv]).reshape(NW, CP, C)
    dst3 = jnp.concatenate([dst, padv]).reshape(NW, CP, C)
    feat_p = jnp.pad(feat, ((0, NP - N), (0, 0)))

    deg_parts = _deg_call()(src3, dst3)
    deg_t = deg_parts.transpose(2, 0, 1).reshape(NP, 4)
    m1, ns, nd = _norm_m1_call(deg_t, feat_p)

    q1 = _agg_call()(m1, src3, dst3)
    h1p, m2 = _mid_call(q1, ns, nd, W1, b1.reshape(1, HID), W2)

    q2 = _agg_call()(m2, src3, dst3)
    h2p = _fin_call(q2, nd, b2.reshape(1, OUT))

    sig = _sig_call(cls_spec_avg_feats, W_sigma)[:, 0]

    h1o = h1p[:N]
    h2o = h2p[:N]
    return (h2o, h1o, h2o, sig)


# speedup vs baseline: 14.6225x; 4.0968x over previous
import functools

import jax
import jax.numpy as jnp
from jax import lax
from jax.experimental import pallas as pl
from jax.experimental.pallas import tpu as pltpu
from jax.experimental.pallas import tpu_sc as plsc

N = 10000
E = 320000
IN_DIM = 128
HID = 256
OUT = 128

NC = 2
NS = 16
NW = NC * NS
C = 128
CP = 80
EPT = C * CP
EP = NW * EPT
NP = 10240
RPT = NP // NS

BR = 1280
GRID = NP // BR

_MESH = dict(core_axis_name="c", subcore_axis_name="s", num_cores=NC,
             num_subcores=NS)


def _deg_body(src_hbm, dst_hbm, out_hbm, src_v, dst_v, ones_v, zer_v,
              dego_sh, degi_sh):
    c = lax.axis_index("c")
    s = lax.axis_index("s")
    wid = c * NS + s
    pltpu.sync_copy(src_hbm.at[wid], src_v)
    pltpu.sync_copy(dst_hbm.at[wid], dst_v)

    def fill_ones(i, _):
        ones_v[pl.ds(i * 16, 16)] = jnp.full((16,), 1.0, jnp.float32)
        return 0
    lax.fori_loop(0, C // 16, fill_ones, 0)

    def fill_zero(i, _):
        zer_v[pl.ds(i * 16, 16)] = jnp.zeros((16,), jnp.float32)
        return 0
    lax.fori_loop(0, RPT // 16, fill_zero, 0)

    pltpu.sync_copy(zer_v, dego_sh.at[pl.ds(s * RPT, RPT)])
    pltpu.sync_copy(zer_v, degi_sh.at[pl.ds(s * RPT, RPT)])
    plsc.subcore_barrier()

    def step(j, _):
        pltpu.sync_copy(ones_v, dego_sh.at[src_v.at[j]], add=True)
        pltpu.sync_copy(ones_v, degi_sh.at[dst_v.at[j]], add=True)
        return 0
    lax.fori_loop(0, CP, step, 0)
    plsc.subcore_barrier()

    pltpu.sync_copy(dego_sh.at[pl.ds(s * RPT, RPT)],
                    out_hbm.at[c, 0, pl.ds(s * RPT, RPT)])
    pltpu.sync_copy(degi_sh.at[pl.ds(s * RPT, RPT)],
                    out_hbm.at[c, 1, pl.ds(s * RPT, RPT)])


def _agg_body(tab_hbm, src_hbm, dst_hbm, out_hbm, src_v, dring_v, buf0_v,
              buf1_v, acc_sh, sem0, sem1, isem0, isem1):
    c = lax.axis_index("c")
    s = lax.axis_index("s")
    wid = c * NS + s
    pltpu.sync_copy(src_hbm.at[wid], src_v)

    def zrow(r, _):
        def zcol(k, _):
            buf0_v[r, pl.ds(k * 16, 16)] = jnp.zeros((16,), jnp.float32)
            return 0
        lax.fori_loop(0, IN_DIM // 16, zcol, 0)
        return 0
    lax.fori_loop(0, C, zrow, 0)

    def zcp(k, _):
        pltpu.sync_copy(buf0_v, acc_sh.at[pl.ds(s * RPT + k * C, C)])
        return 0
    lax.fori_loop(0, RPT // C, zcp, 0)
    plsc.subcore_barrier()

    pltpu.async_copy(dst_hbm.at[wid, 0], dring_v.at[0], isem0)
    pltpu.async_copy(tab_hbm.at[src_v.at[0]], buf0_v, sem0)

    def pair(k, _):
        j0 = 2 * k
        pltpu.async_copy(tab_hbm.at[src_v.at[j0 + 1]], buf1_v, sem1)
        pltpu.async_copy(dst_hbm.at[wid, j0 + 1], dring_v.at[1], isem1)
        pltpu.make_async_copy(dst_hbm.at[wid, j0], dring_v.at[0],
                              isem0).wait()
        pltpu.make_async_copy(tab_hbm.at[src_v.at[j0]], buf0_v, sem0).wait()
        pltpu.sync_copy(buf0_v, acc_sh.at[dring_v.at[0]], add=True)

        @pl.when(j0 + 2 < CP)
        def _():
            pltpu.async_copy(tab_hbm.at[src_v.at[j0 + 2]], buf0_v, sem0)
            pltpu.async_copy(dst_hbm.at[wid, j0 + 2], dring_v.at[0], isem0)

        pltpu.make_async_copy(dst_hbm.at[wid, j0 + 1], dring_v.at[1],
                              isem1).wait()
        pltpu.make_async_copy(tab_hbm.at[src_v.at[j0 + 1]], buf1_v,
                              sem1).wait()
        pltpu.sync_copy(buf1_v, acc_sh.at[dring_v.at[1]], add=True)
        return 0
    lax.fori_loop(0, CP // 2, pair, 0)
    plsc.subcore_barrier()

    def cout(k, _):
        pltpu.sync_copy(acc_sh.at[pl.ds(s * RPT + k * C, C)],
                        out_hbm.at[c, pl.ds(s * RPT + k * C, C)])
        return 0
    lax.fori_loop(0, RPT // C, cout, 0)


@functools.cache
def _deg_call():
    return pl.kernel(
        _deg_body,
        out_type=jax.ShapeDtypeStruct((NC, 2, NP), jnp.float32),
        mesh=plsc.VectorSubcoreMesh(**_MESH),
        scratch_types=[
            pltpu.VMEM((CP, C), jnp.int32),
            pltpu.VMEM((CP, C), jnp.int32),
            pltpu.VMEM((C,), jnp.float32),
            pltpu.VMEM((RPT,), jnp.float32),
            pltpu.VMEM_SHARED((NP,), jnp.float32),
            pltpu.VMEM_SHARED((NP,), jnp.float32),
        ],
    )


@functools.cache
def _agg_call():
    return pl.kernel(
        _agg_body,
        out_type=jax.ShapeDtypeStruct((NC, NP, IN_DIM), jnp.float32),
        mesh=plsc.VectorSubcoreMesh(**_MESH),
        scratch_types=[
            pltpu.VMEM((CP, C), jnp.int32),
            pltpu.VMEM((2, C), jnp.int32),
            pltpu.VMEM((C, IN_DIM), jnp.float32),
            pltpu.VMEM((C, IN_DIM), jnp.float32),
            pltpu.VMEM_SHARED((NP, IN_DIM), jnp.float32),
            pltpu.SemaphoreType.DMA,
            pltpu.SemaphoreType.DMA,
            pltpu.SemaphoreType.DMA,
            pltpu.SemaphoreType.DMA,
        ],
    )


def _norm_m1_body(deg_ref, feat_ref, m1_ref, ns_ref, nd_ref):
    d = deg_ref[...]
    deg_o = d[:, 0] + d[:, 2]
    deg_i = d[:, 1] + d[:, 3]
    ns = jnp.where(deg_o > 0, lax.rsqrt(jnp.maximum(deg_o, 1.0)), 0.0)
    nd = jnp.where(deg_i > 0, lax.rsqrt(jnp.maximum(deg_i, 1.0)), 0.0)
    ns_ref[...] = ns[:, None]
    nd_ref[...] = nd[:, None]
    m1_ref[...] = feat_ref[...] * ns[:, None]


_norm_m1_call = pl.pallas_call(
    _norm_m1_body,
    grid=(GRID,),
    in_specs=[
        pl.BlockSpec((BR, 4), lambda i: (i, 0)),
        pl.BlockSpec((BR, IN_DIM), lambda i: (i, 0)),
    ],
    out_specs=[
        pl.BlockSpec((BR, IN_DIM), lambda i: (i, 0)),
        pl.BlockSpec((BR, 1), lambda i: (i, 0)),
        pl.BlockSpec((BR, 1), lambda i: (i, 0)),
    ],
    out_shape=[
        jax.ShapeDtypeStruct((NP, IN_DIM), jnp.float32),
        jax.ShapeDtypeStruct((NP, 1), jnp.float32),
        jax.ShapeDtypeStruct((NP, 1), jnp.float32),
    ],
)


def _mid_body(q_ref, ns_ref, nd_ref, w1_ref, b1_ref, w2_ref, h1_ref, m2_ref):
    a = (q_ref[0] + q_ref[1]) * nd_ref[...]
    h1 = jnp.dot(a, w1_ref[...], preferred_element_type=jnp.float32)
    h1 = jnp.maximum(h1 + b1_ref[...], 0.0)
    h1_ref[...] = h1
    p = jnp.dot(h1, w2_ref[...], preferred_element_type=jnp.float32)
    m2_ref[...] = p * ns_ref[...]


_mid_call = pl.pallas_call(
    _mid_body,
    grid=(GRID,),
    in_specs=[
        pl.BlockSpec((2, BR, IN_DIM), lambda i: (0, i, 0)),
        pl.BlockSpec((BR, 1), lambda i: (i, 0)),
        pl.BlockSpec((BR, 1), lambda i: (i, 0)),
        pl.BlockSpec((IN_DIM, HID), lambda i: (0, 0)),
        pl.BlockSpec((1, HID), lambda i: (0, 0)),
        pl.BlockSpec((HID, OUT), lambda i: (0, 0)),
    ],
    out_specs=[
        pl.BlockSpec((BR, HID), lambda i: (i, 0)),
        pl.BlockSpec((BR, OUT), lambda i: (i, 0)),
    ],
    out_shape=[
        jax.ShapeDtypeStruct((NP, HID), jnp.float32),
        jax.ShapeDtypeStruct((NP, OUT), jnp.float32),
    ],
)


def _fin_body(q_ref, nd_ref, b2_ref, h2_ref):
    h2_ref[...] = (q_ref[0] + q_ref[1]) * nd_ref[...] + b2_ref[...]


_fin_call = pl.pallas_call(
    _fin_body,
    grid=(GRID,),
    in_specs=[
        pl.BlockSpec((2, BR, OUT), lambda i: (0, i, 0)),
        pl.BlockSpec((BR, 1), lambda i: (i, 0)),
        pl.BlockSpec((1, OUT), lambda i: (0, 0)),
    ],
    out_specs=pl.BlockSpec((BR, OUT), lambda i: (i, 0)),
    out_shape=jax.ShapeDtypeStruct((NP, OUT), jnp.float32),
)


def _sig_body(cls_ref, ws_ref, out_ref):
    z = jnp.dot(cls_ref[...], ws_ref[...], preferred_element_type=jnp.float32)
    out_ref[...] = jnp.maximum(z, 0.0) + jnp.log(1.0 + jnp.exp(-jnp.abs(z)))


_sig_call = pl.pallas_call(
    _sig_body,
    out_shape=jax.ShapeDtypeStruct((10, 1), jnp.float32),
)


def kernel(g, feat, cls_spec_avg_feats, W1, b1, W2, b2, W_sigma):
    src, dst = g[0], g[1]
    padv = N + (jnp.arange(EP - E, dtype=jnp.int32) % (NP - N))
    src3 = jnp.concatenate([src, padv]).reshape(NW, CP, C)
    dst3 = jnp.concatenate([dst, padv]).reshape(NW, CP, C)
    feat_p = jnp.pad(feat, ((0, NP - N), (0, 0)))

    deg_parts = _deg_call()(src3, dst3)
    deg_t = deg_parts.transpose(2, 0, 1).reshape(NP, 4)
    m1, ns, nd = _norm_m1_call(deg_t, feat_p)

    q1 = _agg_call()(m1, src3, dst3)
    h1p, m2 = _mid_call(q1, ns, nd, W1, b1.reshape(1, HID), W2)

    q2 = _agg_call()(m2, src3, dst3)
    h2p = _fin_call(q2, nd, b2.reshape(1, OUT))

    sig = _sig_call(cls_spec_avg_feats, W_sigma)[:, 0]

    h1o = h1p[:N]
    h2o = h2p[:N]
    return (h2o, h1o, h2o, sig)


# speedup vs baseline: 15.3378x; 1.0489x over previous
import functools

import jax
import jax.numpy as jnp
from jax import lax
from jax.experimental import pallas as pl
from jax.experimental.pallas import tpu as pltpu
from jax.experimental.pallas import tpu_sc as plsc

N = 10000
E = 320000
IN_DIM = 128
HID = 256
OUT = 128

NC = 2
NS = 16
NW = NC * NS
C = 128
CP = 80
EPT = C * CP
EP = NW * EPT
NP = 10240
RPT = NP // NS

BR = 1280
GRID = NP // BR

_MESH = dict(core_axis_name="c", subcore_axis_name="s", num_cores=NC,
             num_subcores=NS)


def _deg_body(src_hbm, dst_hbm, out_hbm, src_v, dst_v, ones_v, zer_v,
              dego_sh, degi_sh, osem, isem):
    c = lax.axis_index("c")
    s = lax.axis_index("s")
    wid = c * NS + s
    pltpu.sync_copy(src_hbm.at[wid], src_v)
    pltpu.sync_copy(dst_hbm.at[wid], dst_v)

    def fill_ones(i, _):
        ones_v[pl.ds(i * 16, 16)] = jnp.full((16,), 1.0, jnp.float32)
        return 0
    lax.fori_loop(0, C // 16, fill_ones, 0)

    def fill_zero(i, _):
        zer_v[pl.ds(i * 16, 16)] = jnp.zeros((16,), jnp.float32)
        return 0
    lax.fori_loop(0, RPT // 16, fill_zero, 0)

    pltpu.sync_copy(zer_v, dego_sh.at[pl.ds(s * RPT, RPT)])
    pltpu.sync_copy(zer_v, degi_sh.at[pl.ds(s * RPT, RPT)])
    plsc.subcore_barrier()

    def step(j, _):
        d0 = pltpu.async_copy(ones_v, dego_sh.at[src_v.at[j]], osem,
                              add=True)
        d1 = pltpu.async_copy(ones_v, degi_sh.at[dst_v.at[j]], isem,
                              add=True)
        d0.wait()
        d1.wait()
        return 0
    lax.fori_loop(0, CP, step, 0)
    plsc.subcore_barrier()

    pltpu.sync_copy(dego_sh.at[pl.ds(s * RPT, RPT)],
                    out_hbm.at[c, 0, pl.ds(s * RPT, RPT)])
    pltpu.sync_copy(degi_sh.at[pl.ds(s * RPT, RPT)],
                    out_hbm.at[c, 1, pl.ds(s * RPT, RPT)])


def _agg_body(tab_hbm, src_hbm, dst_hbm, out_hbm, src_v, dring_v, buf0_v,
              buf1_v, acc_sh, sem0, sem1, isem0, isem1):
    c = lax.axis_index("c")
    s = lax.axis_index("s")
    wid = c * NS + s
    pltpu.sync_copy(src_hbm.at[wid], src_v)

    def zrow(r, _):
        def zcol(k, _):
            buf0_v[r, pl.ds(k * 16, 16)] = jnp.zeros((16,), jnp.float32)
            return 0
        lax.fori_loop(0, IN_DIM // 16, zcol, 0)
        return 0
    lax.fori_loop(0, C, zrow, 0)

    def zcp(k, _):
        pltpu.sync_copy(buf0_v, acc_sh.at[pl.ds(s * RPT + k * C, C)])
        return 0
    lax.fori_loop(0, RPT // C, zcp, 0)
    plsc.subcore_barrier()

    pltpu.async_copy(dst_hbm.at[wid, 0], dring_v.at[0], isem0)
    pltpu.async_copy(tab_hbm.at[src_v.at[0]], buf0_v, sem0)

    def pair(k, _):
        j0 = 2 * k
        pltpu.async_copy(tab_hbm.at[src_v.at[j0 + 1]], buf1_v, sem1)
        pltpu.async_copy(dst_hbm.at[wid, j0 + 1], dring_v.at[1], isem1)
        pltpu.make_async_copy(dst_hbm.at[wid, j0], dring_v.at[0],
                              isem0).wait()
        pltpu.make_async_copy(tab_hbm.at[src_v.at[j0]], buf0_v, sem0).wait()
        pltpu.sync_copy(buf0_v, acc_sh.at[dring_v.at[0]], add=True)

        @pl.when(j0 + 2 < CP)
        def _():
            pltpu.async_copy(tab_hbm.at[src_v.at[j0 + 2]], buf0_v, sem0)
            pltpu.async_copy(dst_hbm.at[wid, j0 + 2], dring_v.at[0], isem0)

        pltpu.make_async_copy(dst_hbm.at[wid, j0 + 1], dring_v.at[1],
                              isem1).wait()
        pltpu.make_async_copy(tab_hbm.at[src_v.at[j0 + 1]], buf1_v,
                              sem1).wait()
        pltpu.sync_copy(buf1_v, acc_sh.at[dring_v.at[1]], add=True)
        return 0
    lax.fori_loop(0, CP // 2, pair, 0)
    plsc.subcore_barrier()

    def cout(k, _):
        pltpu.sync_copy(acc_sh.at[pl.ds(s * RPT + k * C, C)],
                        out_hbm.at[c, pl.ds(s * RPT + k * C, C)])
        return 0
    lax.fori_loop(0, RPT // C, cout, 0)


@functools.cache
def _deg_call():
    return pl.kernel(
        _deg_body,
        out_type=jax.ShapeDtypeStruct((NC, 2, NP), jnp.float32),
        mesh=plsc.VectorSubcoreMesh(**_MESH),
        scratch_types=[
            pltpu.VMEM((CP, C), jnp.int32),
            pltpu.VMEM((CP, C), jnp.int32),
            pltpu.VMEM((C,), jnp.float32),
            pltpu.VMEM((RPT,), jnp.float32),
            pltpu.VMEM_SHARED((NP,), jnp.float32),
            pltpu.VMEM_SHARED((NP,), jnp.float32),
            pltpu.SemaphoreType.DMA,
            pltpu.SemaphoreType.DMA,
        ],
    )


@functools.cache
def _agg_call():
    return pl.kernel(
        _agg_body,
        out_type=jax.ShapeDtypeStruct((NC, NP, IN_DIM), jnp.float32),
        mesh=plsc.VectorSubcoreMesh(**_MESH),
        scratch_types=[
            pltpu.VMEM((CP, C), jnp.int32),
            pltpu.VMEM((2, C), jnp.int32),
            pltpu.VMEM((C, IN_DIM), jnp.float32),
            pltpu.VMEM((C, IN_DIM), jnp.float32),
            pltpu.VMEM_SHARED((NP, IN_DIM), jnp.float32),
            pltpu.SemaphoreType.DMA,
            pltpu.SemaphoreType.DMA,
            pltpu.SemaphoreType.DMA,
            pltpu.SemaphoreType.DMA,
        ],
    )


def _norm_m1_body(deg_ref, feat_ref, m1_ref, ns_ref, nd_ref):
    d = deg_ref[...]
    deg_o = d[0] + d[2]
    deg_i = d[1] + d[3]
    ns = jnp.where(deg_o > 0, lax.rsqrt(jnp.maximum(deg_o, 1.0)), 0.0)
    nd = jnp.where(deg_i > 0, lax.rsqrt(jnp.maximum(deg_i, 1.0)), 0.0)
    ns_ref[...] = ns[:, None]
    nd_ref[...] = nd[:, None]
    m1_ref[...] = feat_ref[...] * ns[:, None]


_norm_m1_call = pl.pallas_call(
    _norm_m1_body,
    grid=(GRID,),
    in_specs=[
        pl.BlockSpec((4, BR), lambda i: (0, i)),
        pl.BlockSpec((BR, IN_DIM), lambda i: (i, 0)),
    ],
    out_specs=[
        pl.BlockSpec((BR, IN_DIM), lambda i: (i, 0)),
        pl.BlockSpec((BR, 1), lambda i: (i, 0)),
        pl.BlockSpec((BR, 1), lambda i: (i, 0)),
    ],
    out_shape=[
        jax.ShapeDtypeStruct((NP, IN_DIM), jnp.float32),
        jax.ShapeDtypeStruct((NP, 1), jnp.float32),
        jax.ShapeDtypeStruct((NP, 1), jnp.float32),
    ],
)


def _mid_body(q_ref, ns_ref, nd_ref, w1_ref, b1_ref, w2_ref, h1_ref, m2_ref):
    a = (q_ref[0] + q_ref[1]) * nd_ref[...]
    h1 = jnp.dot(a, w1_ref[...], preferred_element_type=jnp.float32)
    h1 = jnp.maximum(h1 + b1_ref[...], 0.0)
    h1_ref[...] = h1
    p = jnp.dot(h1, w2_ref[...], preferred_element_type=jnp.float32)
    m2_ref[...] = p * ns_ref[...]


_mid_call = pl.pallas_call(
    _mid_body,
    grid=(GRID,),
    in_specs=[
        pl.BlockSpec((2, BR, IN_DIM), lambda i: (0, i, 0)),
        pl.BlockSpec((BR, 1), lambda i: (i, 0)),
        pl.BlockSpec((BR, 1), lambda i: (i, 0)),
        pl.BlockSpec((IN_DIM, HID), lambda i: (0, 0)),
        pl.BlockSpec((1, HID), lambda i: (0, 0)),
        pl.BlockSpec((HID, OUT), lambda i: (0, 0)),
    ],
    out_specs=[
        pl.BlockSpec((BR, HID), lambda i: (i, 0)),
        pl.BlockSpec((BR, OUT), lambda i: (i, 0)),
    ],
    out_shape=[
        jax.ShapeDtypeStruct((N, HID), jnp.float32),
        jax.ShapeDtypeStruct((NP, OUT), jnp.float32),
    ],
)


def _fin_body(q_ref, nd_ref, b2_ref, cls_ref, ws_ref, h2_ref, sig_ref):
    h2_ref[...] = (q_ref[0] + q_ref[1]) * nd_ref[...] + b2_ref[...]

    @pl.when(pl.program_id(0) == 0)
    def _():
        z = jnp.dot(cls_ref[...], ws_ref[...],
                    preferred_element_type=jnp.float32)
        sig_ref[...] = jnp.maximum(z, 0.0) + jnp.log(1.0 + jnp.exp(-jnp.abs(z)))


_fin_call = pl.pallas_call(
    _fin_body,
    grid=(GRID,),
    in_specs=[
        pl.BlockSpec((2, BR, OUT), lambda i: (0, i, 0)),
        pl.BlockSpec((BR, 1), lambda i: (i, 0)),
        pl.BlockSpec((1, OUT), lambda i: (0, 0)),
        pl.BlockSpec((10, IN_DIM), lambda i: (0, 0)),
        pl.BlockSpec((IN_DIM, 1), lambda i: (0, 0)),
    ],
    out_specs=[
        pl.BlockSpec((BR, OUT), lambda i: (i, 0)),
        pl.BlockSpec((10, 1), lambda i: (0, 0)),
    ],
    out_shape=[
        jax.ShapeDtypeStruct((N, OUT), jnp.float32),
        jax.ShapeDtypeStruct((10, 1), jnp.float32),
    ],
)


def kernel(g, feat, cls_spec_avg_feats, W1, b1, W2, b2, W_sigma):
    src, dst = g[0], g[1]
    padv = N + (jnp.arange(EP - E, dtype=jnp.int32) % (NP - N))
    src3 = jnp.concatenate([src, padv]).reshape(NW, CP, C)
    dst3 = jnp.concatenate([dst, padv]).reshape(NW, CP, C)

    deg_parts = _deg_call()(src3, dst3)
    deg_f = deg_parts.reshape(4, NP)
    m1, ns, nd = _norm_m1_call(deg_f, feat)

    q1 = _agg_call()(m1, src3, dst3)
    h1o, m2 = _mid_call(q1, ns, nd, W1, b1.reshape(1, HID), W2)

    q2 = _agg_call()(m2, src3, dst3)
    h2o, sig2 = _fin_call(q2, nd, b2.reshape(1, OUT), cls_spec_avg_feats,
                          W_sigma)
    return (h2o, h1o, h2o, sig2[:, 0])
